# baseline (speedup 1.0000x reference)
import functools

import jax
import jax.numpy as jnp
from jax import lax
from jax.experimental import pallas as pl
from jax.experimental.pallas import tpu as pltpu
from jax.experimental.pallas import tpu_sc as plsc

MIN_NORM = 1e-15
EPS = 4e-3

NC = 2
NS = 16
NW = NC * NS
LANES = 16
CHUNK = 128


def _artanh(z):
    z = jnp.clip(z, -1.0 + 1e-7, 1.0 - 1e-7)
    return 0.5 * jnp.log((1.0 + z) / (1.0 - z))


def _rnorm(v):
    return jnp.maximum(jnp.sqrt(jnp.sum(v * v, axis=-1, keepdims=True)), MIN_NORM)


def _proj(v):
    n = _rnorm(v)
    maxnorm = 1.0 - EPS
    return jnp.where(n > maxnorm, v / n * maxnorm, v)


def _expmap0(u):
    n = _rnorm(u)
    return jnp.tanh(n) * u / n


def _logmap0(p):
    n = _rnorm(p)
    return _artanh(n) * p / n


def _lin_body(x_ref, w_ref, b_ref, xt_ref):
    xb = x_ref[...]
    w = w_ref[...]
    b = b_ref[...]
    xn = _rnorm(xb)
    mx = lax.dot_general(xb, w, (((1,), (1,)), ((), ())),
                         preferred_element_type=jnp.float32)
    mxn = _rnorm(mx)
    res = jnp.tanh(mxn / xn * _artanh(xn)) * mx / mxn
    zero_rows = jnp.max(jnp.abs(mx), axis=-1, keepdims=True) == 0.0
    res = jnp.where(zero_rows, jnp.zeros_like(res), res)
    res = _proj(res)
    hb = _proj(_expmap0(b))
    r2 = jnp.sum(res * res, axis=-1, keepdims=True)
    h2 = jnp.sum(hb * hb, axis=-1, keepdims=True)
    rh = jnp.sum(res * hb, axis=-1, keepdims=True)
    num = (1.0 + 2.0 * rh + h2) * res + (1.0 - r2) * hb
    den = jnp.maximum(1.0 + 2.0 * rh + r2 * h2, MIN_NORM)
    h = _proj(num / den)
    xt_ref[...] = _logmap0(h)


def _scale_body(parts_ref, xt_ref, y_ref):
    deg = 1.0 + jnp.sum(parts_ref[...], axis=1)
    dis = lax.rsqrt(deg)[:, None]
    y_ref[...] = dis * xt_ref[...]


def _post_body(acc_ref, parts_ref, xt_ref, out_ref):
    acc = acc_ref[0] + acc_ref[1]
    deg = 1.0 + jnp.sum(parts_ref[...], axis=1)
    dis = lax.rsqrt(deg)[:, None]
    xt = xt_ref[...]
    support = dis * acc + (dis * dis) * xt
    out1 = _proj(_expmap0(support))
    lt = _logmap0(out1)
    act = jnp.where(lt >= 0.0, lt, 0.01 * lt)
    out_ref[...] = _proj(_expmap0(act))


def _deg_body(n_nodes, per_tile, row_hbm, col_hbm, parts_hbm, colp_hbm,
              rbuf, cbuf, cpbuf, degbuf):
    wid = lax.axis_index("s") * NC + lax.axis_index("c")
    base = wid * per_tile
    pltpu.sync_copy(row_hbm.at[pl.ds(base, per_tile)], rbuf)
    pltpu.sync_copy(col_hbm.at[pl.ds(base, per_tile)], cbuf)

    def zero_body(i, carry):
        degbuf[pl.ds(i * LANES, LANES)] = jnp.zeros((LANES,), jnp.float32)
        return carry

    lax.fori_loop(0, n_nodes // LANES, zero_body, 0)

    ones = jnp.full((LANES,), 1.0, jnp.float32)
    npad = jnp.full((LANES,), n_nodes, jnp.int32)

    def edge_body(i, carry):
        r = rbuf[pl.ds(i * LANES, LANES)]
        c = cbuf[pl.ds(i * LANES, LANES)]
        m = r != c
        cpbuf[pl.ds(i * LANES, LANES)] = jnp.where(m, c, npad)
        plsc.addupdate_scatter(degbuf, [r], ones, mask=m)
        return carry

    lax.fori_loop(0, per_tile // LANES, edge_body, 0)

    pltpu.sync_copy(degbuf, parts_hbm.at[wid])
    pltpu.sync_copy(cpbuf, colp_hbm.at[pl.ds(base, per_tile)])


NPH = 2


def _agg_body(n_pad, per_tile, row_hbm, colp_hbm, yp_hbm, z_hbm, out_hbm,
              ridx, cidx, gbuf, acc, gsem, ssem):
    cid = lax.axis_index("c")
    sid = lax.axis_index("s")
    wid = sid * NC + cid
    rows_per_tile = n_pad // NS
    zb = sid * rows_per_tile
    pltpu.sync_copy(z_hbm.at[pl.ds(zb, rows_per_tile)],
                    acc.at[pl.ds(zb, rows_per_tile)])
    plsc.subcore_barrier()

    nch = per_tile // CHUNK
    m = nch // NPH

    def gather_start(g, s):
        pltpu.async_copy(yp_hbm.at[cidx.at[g]], gbuf[s], gsem[s])

    def gather_wait(g, s):
        pltpu.make_async_copy(yp_hbm.at[cidx.at[g]], gbuf[s], gsem[s]).wait()

    def scatter_start(g, s):
        pltpu.async_copy(gbuf[s], acc.at[ridx.at[g]], ssem[s], add=True)

    def scatter_wait(g, s):
        pltpu.make_async_copy(gbuf[s], acc.at[ridx.at[g]], ssem[s]).wait()

    for p in range(NPH):
        pltpu.sync_copy(row_hbm.at[wid, pl.ds(p * m, m)], ridx)
        pltpu.sync_copy(colp_hbm.at[wid, pl.ds(p * m, m)], cidx)
        gather_start(0, 0)

        def pair_body(j, carry):
            for s in range(2):
                g = 2 * j + s

                @pl.when(g < m)
                def _():
                    gather_wait(g, s)
                    scatter_start(g, s)

                    @pl.when(g + 1 < m)
                    def _():
                        @pl.when(g >= 1)
                        def _():
                            scatter_wait(g - 1, 1 - s)
                        gather_start(g + 1, 1 - s)

            return carry

        lax.fori_loop(0, m // 2, pair_body, 0)
        scatter_wait(m - 2, m % 2)
        scatter_wait(m - 1, (m - 1) % 2)
    plsc.subcore_barrier()
    pltpu.sync_copy(acc.at[pl.ds(zb, rows_per_tile)],
                    out_hbm.at[cid, pl.ds(zb, rows_per_tile)])


@jax.jit
def kernel(x, edge_index, W, b):
    N, D_in = x.shape
    D = W.shape[0]
    E = edge_index.shape[1]
    row = edge_index[0]
    col = edge_index[1]

    rows_block = 1000
    grid = N // rows_block

    x_t = pl.pallas_call(
        _lin_body,
        grid=(grid,),
        in_specs=[
            pl.BlockSpec((rows_block, D_in), lambda i: (i, 0)),
            pl.BlockSpec((D, D_in), lambda i: (0, 0)),
            pl.BlockSpec((1, D), lambda i: (0, 0)),
        ],
        out_specs=pl.BlockSpec((rows_block, D), lambda i: (i, 0)),
        out_shape=jax.ShapeDtypeStruct((N, D), jnp.float32),
    )(x, W, b.reshape(1, D))

    per_tile_b = E // NW
    mesh = plsc.VectorSubcoreMesh(core_axis_name="c", subcore_axis_name="s")
    sc_params = pltpu.CompilerParams(needs_layout_passes=False)
    deg_kernel = functools.partial(
        pl.kernel,
        mesh=mesh,
        compiler_params=sc_params,
        out_type=[
            jax.ShapeDtypeStruct((NW, N), jnp.float32),
            jax.ShapeDtypeStruct((E,), jnp.int32),
        ],
        scratch_types=[
            pltpu.VMEM((per_tile_b,), jnp.int32),
            pltpu.VMEM((per_tile_b,), jnp.int32),
            pltpu.VMEM((per_tile_b,), jnp.int32),
            pltpu.VMEM((N,), jnp.float32),
        ],
    )(functools.partial(_deg_body, N, per_tile_b))
    parts, colp = deg_kernel(row, col)
    parts_t = parts.T

    y = pl.pallas_call(
        _scale_body,
        grid=(grid,),
        in_specs=[
            pl.BlockSpec((rows_block, NW), lambda i: (i, 0)),
            pl.BlockSpec((rows_block, D), lambda i: (i, 0)),
        ],
        out_specs=pl.BlockSpec((rows_block, D), lambda i: (i, 0)),
        out_shape=jax.ShapeDtypeStruct((N, D), jnp.float32),
    )(parts_t, x_t)

    chgrp = 2 * NPH * CHUNK
    per_tile_d = ((E // NW + chgrp - 1) // chgrp) * chgrp
    EP = per_tile_d * NW
    NP = ((N + 8 * NS - 1) // (8 * NS)) * (8 * NS)
    y_pad = jnp.concatenate([y, jnp.zeros((1, D), jnp.float32)], axis=0)
    nch_d = per_tile_d // CHUNK
    rowp = jnp.concatenate(
        [row, jnp.zeros((EP - E,), jnp.int32)]).reshape(NW, nch_d, CHUNK)
    colpp = jnp.concatenate(
        [colp, jnp.full((EP - E,), N, jnp.int32)]).reshape(NW, nch_d, CHUNK)
    z = jnp.zeros((NP, D), jnp.float32)

    agg_kernel = functools.partial(
        pl.kernel,
        mesh=mesh,
        compiler_params=sc_params,
        out_type=jax.ShapeDtypeStruct((NC, NP, D), jnp.float32),
        scratch_types=[
            pltpu.VMEM((nch_d // NPH, CHUNK), jnp.int32),
            pltpu.VMEM((nch_d // NPH, CHUNK), jnp.int32),
            [pltpu.VMEM((CHUNK, D), jnp.float32) for _ in range(2)],
            pltpu.VMEM_SHARED((NP, D), jnp.float32),
            [pltpu.SemaphoreType.DMA for _ in range(2)],
            [pltpu.SemaphoreType.DMA for _ in range(2)],
        ],
    )(functools.partial(_agg_body, NP, per_tile_d))
    acc2 = agg_kernel(rowp, colpp, y_pad, z)

    out = pl.pallas_call(
        _post_body,
        grid=(grid,),
        in_specs=[
            pl.BlockSpec((NC, rows_block, D), lambda i: (0, i, 0)),
            pl.BlockSpec((rows_block, NW), lambda i: (i, 0)),
            pl.BlockSpec((rows_block, D), lambda i: (i, 0)),
        ],
        out_specs=pl.BlockSpec((rows_block, D), lambda i: (i, 0)),
        out_shape=jax.ShapeDtypeStruct((N, D), jnp.float32),
    )(acc2, parts_t, x_t)
    return out


# speedup vs baseline: 1.3418x; 1.3418x over previous
import functools

import jax
import jax.numpy as jnp
from jax import lax
from jax.experimental import pallas as pl
from jax.experimental.pallas import tpu as pltpu
from jax.experimental.pallas import tpu_sc as plsc

MIN_NORM = 1e-15
EPS = 4e-3

NC = 2
NS = 16
NW = NC * NS
LANES = 16
CHUNK = 128


def _artanh(z):
    z = jnp.clip(z, -1.0 + 1e-7, 1.0 - 1e-7)
    return 0.5 * jnp.log((1.0 + z) / (1.0 - z))


def _rnorm(v):
    return jnp.maximum(jnp.sqrt(jnp.sum(v * v, axis=-1, keepdims=True)), MIN_NORM)


def _proj(v):
    n = _rnorm(v)
    maxnorm = 1.0 - EPS
    return jnp.where(n > maxnorm, v / n * maxnorm, v)


def _expmap0(u):
    n = _rnorm(u)
    return jnp.tanh(n) * u / n


def _logmap0(p):
    n = _rnorm(p)
    return _artanh(n) * p / n


def _lin_body(x_ref, w_ref, b_ref, xt_ref):
    xb = x_ref[...]
    w = w_ref[...]
    b = b_ref[...]
    xn = _rnorm(xb)
    mx = lax.dot_general(xb, w, (((1,), (1,)), ((), ())),
                         preferred_element_type=jnp.float32)
    mxn = _rnorm(mx)
    res = jnp.tanh(mxn / xn * _artanh(xn)) * mx / mxn
    zero_rows = jnp.max(jnp.abs(mx), axis=-1, keepdims=True) == 0.0
    res = jnp.where(zero_rows, jnp.zeros_like(res), res)
    res = _proj(res)
    hb = _proj(_expmap0(b))
    r2 = jnp.sum(res * res, axis=-1, keepdims=True)
    h2 = jnp.sum(hb * hb, axis=-1, keepdims=True)
    rh = jnp.sum(res * hb, axis=-1, keepdims=True)
    num = (1.0 + 2.0 * rh + h2) * res + (1.0 - r2) * hb
    den = jnp.maximum(1.0 + 2.0 * rh + r2 * h2, MIN_NORM)
    h = _proj(num / den)
    xt_ref[...] = _logmap0(h)


def _scale_body(parts_ref, xt_ref, y_ref):
    deg = 1.0 + jnp.sum(parts_ref[...], axis=1)
    dis = lax.rsqrt(deg)[:, None]
    y_ref[...] = dis * xt_ref[...]


def _post_body(acc_ref, parts_ref, xt_ref, out_ref):
    acc = acc_ref[0] + acc_ref[1]
    deg = 1.0 + jnp.sum(parts_ref[...], axis=1)
    dis = lax.rsqrt(deg)[:, None]
    xt = xt_ref[...]
    support = dis * acc + (dis * dis) * xt
    out1 = _proj(_expmap0(support))
    lt = _logmap0(out1)
    act = jnp.where(lt >= 0.0, lt, 0.01 * lt)
    out_ref[...] = _proj(_expmap0(act))


def _deg_body(n_nodes, per_tile, row_hbm, col_hbm, parts_hbm, colp_hbm,
              rbuf, cbuf, cpbuf, degbuf):
    wid = lax.axis_index("s") * NC + lax.axis_index("c")
    base = wid * per_tile
    pltpu.sync_copy(row_hbm.at[pl.ds(base, per_tile)], rbuf)
    pltpu.sync_copy(col_hbm.at[pl.ds(base, per_tile)], cbuf)

    def zero_body(i, carry):
        degbuf[pl.ds(i * LANES, LANES)] = jnp.zeros((LANES,), jnp.float32)
        return carry

    lax.fori_loop(0, n_nodes // LANES, zero_body, 0)

    ones = jnp.full((LANES,), 1.0, jnp.float32)
    npad = jnp.full((LANES,), n_nodes, jnp.int32)

    def edge_body(i, carry):
        r = rbuf[pl.ds(i * LANES, LANES)]
        c = cbuf[pl.ds(i * LANES, LANES)]
        m = r != c
        cpbuf[pl.ds(i * LANES, LANES)] = jnp.where(m, c, npad)
        plsc.addupdate_scatter(degbuf, [r], ones, mask=m)
        return carry

    lax.fori_loop(0, per_tile // LANES, edge_body, 0)

    pltpu.sync_copy(degbuf, parts_hbm.at[wid])
    pltpu.sync_copy(cpbuf, colp_hbm.at[pl.ds(base, per_tile)])


def _agg_body(n_pad, per_tile, row_hbm, colp_hbm, yp_hbm, z_hbm, out_hbm,
              ridx, cidx, gbuf, acc, gsem):
    cid = lax.axis_index("c")
    sid = lax.axis_index("s")
    wid = sid * NC + cid
    rows_per_tile = n_pad // NS
    zb = sid * rows_per_tile
    pltpu.sync_copy(z_hbm.at[pl.ds(zb, rows_per_tile)],
                    acc.at[pl.ds(zb, rows_per_tile)])
    plsc.subcore_barrier()

    pltpu.sync_copy(row_hbm.at[wid], ridx)
    pltpu.sync_copy(colp_hbm.at[wid], cidx)

    def chunk_body(g, carry):
        pltpu.async_copy(yp_hbm.at[cidx.at[g]], gbuf, gsem).wait()
        pltpu.sync_copy(gbuf, acc.at[ridx.at[g]], add=True)
        return carry

    lax.fori_loop(0, per_tile // CHUNK, chunk_body, 0)
    plsc.subcore_barrier()
    pltpu.sync_copy(acc.at[pl.ds(zb, rows_per_tile)],
                    out_hbm.at[cid, pl.ds(zb, rows_per_tile)])


@jax.jit
def kernel(x, edge_index, W, b):
    N, D_in = x.shape
    D = W.shape[0]
    E = edge_index.shape[1]
    row = edge_index[0]
    col = edge_index[1]

    rows_block = 1000
    grid = N // rows_block

    x_t = pl.pallas_call(
        _lin_body,
        grid=(grid,),
        in_specs=[
            pl.BlockSpec((rows_block, D_in), lambda i: (i, 0)),
            pl.BlockSpec((D, D_in), lambda i: (0, 0)),
            pl.BlockSpec((1, D), lambda i: (0, 0)),
        ],
        out_specs=pl.BlockSpec((rows_block, D), lambda i: (i, 0)),
        out_shape=jax.ShapeDtypeStruct((N, D), jnp.float32),
    )(x, W, b.reshape(1, D))

    per_tile_b = E // NW
    mesh = plsc.VectorSubcoreMesh(core_axis_name="c", subcore_axis_name="s")
    sc_params = pltpu.CompilerParams(needs_layout_passes=False)
    deg_kernel = functools.partial(
        pl.kernel,
        mesh=mesh,
        compiler_params=sc_params,
        out_type=[
            jax.ShapeDtypeStruct((NW, N), jnp.float32),
            jax.ShapeDtypeStruct((E,), jnp.int32),
        ],
        scratch_types=[
            pltpu.VMEM((per_tile_b,), jnp.int32),
            pltpu.VMEM((per_tile_b,), jnp.int32),
            pltpu.VMEM((per_tile_b,), jnp.int32),
            pltpu.VMEM((N,), jnp.float32),
        ],
    )(functools.partial(_deg_body, N, per_tile_b))
    parts, colp = deg_kernel(row, col)
    parts_t = parts.T

    y = pl.pallas_call(
        _scale_body,
        grid=(grid,),
        in_specs=[
            pl.BlockSpec((rows_block, NW), lambda i: (i, 0)),
            pl.BlockSpec((rows_block, D), lambda i: (i, 0)),
        ],
        out_specs=pl.BlockSpec((rows_block, D), lambda i: (i, 0)),
        out_shape=jax.ShapeDtypeStruct((N, D), jnp.float32),
    )(parts_t, x_t)

    per_tile_d = ((E // NW + CHUNK - 1) // CHUNK) * CHUNK
    EP = per_tile_d * NW
    NP = ((N + 8 * NS - 1) // (8 * NS)) * (8 * NS)
    y_pad = jnp.concatenate([y, jnp.zeros((1, D), jnp.float32)], axis=0)
    nch_d = per_tile_d // CHUNK
    rowp = jnp.concatenate(
        [row, jnp.zeros((EP - E,), jnp.int32)]).reshape(NW, nch_d, CHUNK)
    colpp = jnp.concatenate(
        [colp, jnp.full((EP - E,), N, jnp.int32)]).reshape(NW, nch_d, CHUNK)
    z = jnp.zeros((NP, D), jnp.float32)

    agg_kernel = functools.partial(
        pl.kernel,
        mesh=mesh,
        compiler_params=sc_params,
        out_type=jax.ShapeDtypeStruct((NC, NP, D), jnp.float32),
        scratch_types=[
            pltpu.VMEM((nch_d, CHUNK), jnp.int32),
            pltpu.VMEM((nch_d, CHUNK), jnp.int32),
            pltpu.VMEM((CHUNK, D), jnp.float32),
            pltpu.VMEM_SHARED((NP, D), jnp.float32),
            pltpu.SemaphoreType.DMA,
        ],
    )(functools.partial(_agg_body, NP, per_tile_d))
    acc2 = agg_kernel(rowp, colpp, y_pad, z)

    out = pl.pallas_call(
        _post_body,
        grid=(grid,),
        in_specs=[
            pl.BlockSpec((NC, rows_block, D), lambda i: (0, i, 0)),
            pl.BlockSpec((rows_block, NW), lambda i: (i, 0)),
            pl.BlockSpec((rows_block, D), lambda i: (i, 0)),
        ],
        out_specs=pl.BlockSpec((rows_block, D), lambda i: (i, 0)),
        out_shape=jax.ShapeDtypeStruct((N, D), jnp.float32),
    )(acc2, parts_t, x_t)
    return out


# speedup vs baseline: 1.5021x; 1.1195x over previous
import functools

import jax
import jax.numpy as jnp
from jax import lax
from jax.experimental import pallas as pl
from jax.experimental.pallas import tpu as pltpu
from jax.experimental.pallas import tpu_sc as plsc

MIN_NORM = 1e-15
EPS = 4e-3

NC = 2
NS = 16
NW = NC * NS
LANES = 16
CHUNK = 64


def _artanh(z):
    z = jnp.clip(z, -1.0 + 1e-7, 1.0 - 1e-7)
    return 0.5 * jnp.log((1.0 + z) / (1.0 - z))


def _rnorm(v):
    return jnp.maximum(jnp.sqrt(jnp.sum(v * v, axis=-1, keepdims=True)), MIN_NORM)


def _proj(v):
    n = _rnorm(v)
    maxnorm = 1.0 - EPS
    return jnp.where(n > maxnorm, v / n * maxnorm, v)


def _expmap0(u):
    n = _rnorm(u)
    return jnp.tanh(n) * u / n


def _logmap0(p):
    n = _rnorm(p)
    return _artanh(n) * p / n


def _lin_body(x_ref, w_ref, b_ref, xt_ref):
    xb = x_ref[...]
    w = w_ref[...]
    b = b_ref[...]
    xn = _rnorm(xb)
    mx = lax.dot_general(xb, w, (((1,), (1,)), ((), ())),
                         preferred_element_type=jnp.float32)
    mxn = _rnorm(mx)
    res = jnp.tanh(mxn / xn * _artanh(xn)) * mx / mxn
    zero_rows = jnp.max(jnp.abs(mx), axis=-1, keepdims=True) == 0.0
    res = jnp.where(zero_rows, jnp.zeros_like(res), res)
    res = _proj(res)
    hb = _proj(_expmap0(b))
    r2 = jnp.sum(res * res, axis=-1, keepdims=True)
    h2 = jnp.sum(hb * hb, axis=-1, keepdims=True)
    rh = jnp.sum(res * hb, axis=-1, keepdims=True)
    num = (1.0 + 2.0 * rh + h2) * res + (1.0 - r2) * hb
    den = jnp.maximum(1.0 + 2.0 * rh + r2 * h2, MIN_NORM)
    h = _proj(num / den)
    xt_ref[...] = _logmap0(h)


def _scale_body(parts_ref, xt_ref, y_ref):
    deg = 1.0 + jnp.sum(parts_ref[...], axis=1)
    dis = lax.rsqrt(deg)[:, None]
    y_ref[...] = dis * xt_ref[...]


def _post_body(acc_ref, parts_ref, xt_ref, out_ref):
    acc = acc_ref[0] + acc_ref[1]
    deg = 1.0 + jnp.sum(parts_ref[...], axis=1)
    dis = lax.rsqrt(deg)[:, None]
    xt = xt_ref[...]
    support = dis * acc + (dis * dis) * xt
    out1 = _proj(_expmap0(support))
    lt = _logmap0(out1)
    act = jnp.where(lt >= 0.0, lt, 0.01 * lt)
    out_ref[...] = _proj(_expmap0(act))


def _deg_body(n_nodes, per_tile, row_hbm, col_hbm, parts_hbm, colp_hbm,
              rbuf, cbuf, cpbuf, degbuf):
    wid = lax.axis_index("s") * NC + lax.axis_index("c")
    base = wid * per_tile
    pltpu.sync_copy(row_hbm.at[pl.ds(base, per_tile)], rbuf)
    pltpu.sync_copy(col_hbm.at[pl.ds(base, per_tile)], cbuf)

    def zero_body(i, carry):
        degbuf[pl.ds(i * LANES, LANES)] = jnp.zeros((LANES,), jnp.float32)
        return carry

    lax.fori_loop(0, n_nodes // LANES, zero_body, 0)

    ones = jnp.full((LANES,), 1.0, jnp.float32)
    npad = jnp.full((LANES,), n_nodes, jnp.int32)

    def edge_body(i, carry):
        r = rbuf[pl.ds(i * LANES, LANES)]
        c = cbuf[pl.ds(i * LANES, LANES)]
        m = r != c
        cpbuf[pl.ds(i * LANES, LANES)] = jnp.where(m, c, npad)
        plsc.addupdate_scatter(degbuf, [r], ones, mask=m)
        return carry

    lax.fori_loop(0, per_tile // LANES, edge_body, 0)

    pltpu.sync_copy(degbuf, parts_hbm.at[wid])
    pltpu.sync_copy(cpbuf, colp_hbm.at[pl.ds(base, per_tile)])


def _agg_body(n_pad, per_tile, row_hbm, colp_hbm, yp_hbm, z_hbm, out_hbm,
              ridx, cidx, gbuf, acc, gsem):
    cid = lax.axis_index("c")
    sid = lax.axis_index("s")
    wid = sid * NC + cid
    rows_per_tile = n_pad // NS
    zb = sid * rows_per_tile
    pltpu.sync_copy(z_hbm.at[pl.ds(zb, rows_per_tile)],
                    acc.at[pl.ds(zb, rows_per_tile)])
    plsc.subcore_barrier()

    pltpu.sync_copy(row_hbm.at[wid], ridx)
    pltpu.sync_copy(colp_hbm.at[wid], cidx)

    def chunk_body(g, carry):
        pltpu.async_copy(yp_hbm.at[cidx.at[g]], gbuf, gsem).wait()
        pltpu.sync_copy(gbuf, acc.at[ridx.at[g]], add=True)
        return carry

    lax.fori_loop(0, per_tile // CHUNK, chunk_body, 0)
    plsc.subcore_barrier()
    pltpu.sync_copy(acc.at[pl.ds(zb, rows_per_tile)],
                    out_hbm.at[cid, pl.ds(zb, rows_per_tile)])


@jax.jit
def kernel(x, edge_index, W, b):
    N, D_in = x.shape
    D = W.shape[0]
    E = edge_index.shape[1]
    row = edge_index[0]
    col = edge_index[1]

    rows_block = 1000
    grid = N // rows_block

    x_t = pl.pallas_call(
        _lin_body,
        grid=(grid,),
        in_specs=[
            pl.BlockSpec((rows_block, D_in), lambda i: (i, 0)),
            pl.BlockSpec((D, D_in), lambda i: (0, 0)),
            pl.BlockSpec((1, D), lambda i: (0, 0)),
        ],
        out_specs=pl.BlockSpec((rows_block, D), lambda i: (i, 0)),
        out_shape=jax.ShapeDtypeStruct((N, D), jnp.float32),
    )(x, W, b.reshape(1, D))

    per_tile_b = E // NW
    mesh = plsc.VectorSubcoreMesh(core_axis_name="c", subcore_axis_name="s")
    sc_params = pltpu.CompilerParams(needs_layout_passes=False)
    deg_kernel = functools.partial(
        pl.kernel,
        mesh=mesh,
        compiler_params=sc_params,
        out_type=[
            jax.ShapeDtypeStruct((NW, N), jnp.float32),
            jax.ShapeDtypeStruct((E,), jnp.int32),
        ],
        scratch_types=[
            pltpu.VMEM((per_tile_b,), jnp.int32),
            pltpu.VMEM((per_tile_b,), jnp.int32),
            pltpu.VMEM((per_tile_b,), jnp.int32),
            pltpu.VMEM((N,), jnp.float32),
        ],
    )(functools.partial(_deg_body, N, per_tile_b))
    parts, colp = deg_kernel(row, col)
    parts_t = parts.T

    y = pl.pallas_call(
        _scale_body,
        grid=(grid,),
        in_specs=[
            pl.BlockSpec((rows_block, NW), lambda i: (i, 0)),
            pl.BlockSpec((rows_block, D), lambda i: (i, 0)),
        ],
        out_specs=pl.BlockSpec((rows_block, D), lambda i: (i, 0)),
        out_shape=jax.ShapeDtypeStruct((N, D), jnp.float32),
    )(parts_t, x_t)

    per_tile_d = ((E // NW + CHUNK - 1) // CHUNK) * CHUNK
    EP = per_tile_d * NW
    NP = ((N + 8 * NS - 1) // (8 * NS)) * (8 * NS)
    y_pad = jnp.concatenate([y, jnp.zeros((1, D), jnp.float32)], axis=0)
    nch_d = per_tile_d // CHUNK
    rowp = jnp.concatenate(
        [row, jnp.zeros((EP - E,), jnp.int32)]).reshape(NW, nch_d, CHUNK)
    colpp = jnp.concatenate(
        [colp, jnp.full((EP - E,), N, jnp.int32)]).reshape(NW, nch_d, CHUNK)
    z = jnp.zeros((NP, D), jnp.float32)

    agg_kernel = functools.partial(
        pl.kernel,
        mesh=mesh,
        compiler_params=sc_params,
        out_type=jax.ShapeDtypeStruct((NC, NP, D), jnp.float32),
        scratch_types=[
            pltpu.VMEM((nch_d, CHUNK), jnp.int32),
            pltpu.VMEM((nch_d, CHUNK), jnp.int32),
            pltpu.VMEM((CHUNK, D), jnp.float32),
            pltpu.VMEM_SHARED((NP, D), jnp.float32),
            pltpu.SemaphoreType.DMA,
        ],
    )(functools.partial(_agg_body, NP, per_tile_d))
    acc2 = agg_kernel(rowp, colpp, y_pad, z)

    out = pl.pallas_call(
        _post_body,
        grid=(grid,),
        in_specs=[
            pl.BlockSpec((NC, rows_block, D), lambda i: (0, i, 0)),
            pl.BlockSpec((rows_block, NW), lambda i: (i, 0)),
            pl.BlockSpec((rows_block, D), lambda i: (i, 0)),
        ],
        out_specs=pl.BlockSpec((rows_block, D), lambda i: (i, 0)),
        out_shape=jax.ShapeDtypeStruct((N, D), jnp.float32),
    )(acc2, parts_t, x_t)
    return out
